# baseline (speedup 1.0000x reference)
import functools

import jax
import jax.numpy as jnp
from jax import lax
from jax.experimental import pallas as pl
from jax.experimental.pallas import tpu as pltpu, tpu_sc as plsc

NUM_CORES = 2
NUM_SUBCORES = 16
NW = NUM_CORES * NUM_SUBCORES
L = 16
CH = 256


def _sc_gather_wide(sup2d, table2, b_per_w, W):
    mesh = plsc.VectorSubcoreMesh(core_axis_name="c", subcore_axis_name="s")
    n_chunks = b_per_w // CH

    @functools.partial(
        pl.kernel,
        out_type=jax.ShapeDtypeStruct((NW, b_per_w, W), jnp.float32),
        mesh=mesh,
        scratch_types=[
            pltpu.VMEM((b_per_w,), jnp.int32),
            pltpu.VMEM((CH, W), jnp.float32),
            pltpu.SemaphoreType.DMA,
        ],
    )
    def k(table_hbm, idx_hbm, out_hbm, idx_v, rows_v, sem):
        wid = lax.axis_index("s") * NUM_CORES + lax.axis_index("c")
        pltpu.sync_copy(idx_hbm.at[wid], idx_v)

        def chunk(c, _):
            def group(g, _):
                vec = idx_v[pl.ds(c * CH + g * L, L)]
                for l in range(L):
                    r = jnp.squeeze(lax.slice(vec, (l,), (l + 1,)))
                    pltpu.async_copy(table_hbm.at[r], rows_v.at[g * L + l], sem)
                return 0

            lax.fori_loop(0, CH // L, group, 0)
            pltpu.make_async_copy(
                out_hbm.at[wid, pl.ds(c * CH, CH)], rows_v, sem
            ).wait()
            pltpu.sync_copy(rows_v, out_hbm.at[wid, pl.ds(c * CH, CH)])
            return 0

        lax.fori_loop(0, n_chunks, chunk, 0)

    return k(table2, sup2d)


def kernel(labels, train, table):
    B = labels.shape[0]
    V, D = table.shape
    lab = labels.astype(jnp.int32)
    n = (V // 2) * 2
    table2 = table[:n].reshape(n // 2, 2 * D)
    b_per_w = B // NW
    sup = (lab >> 1).reshape(NW, b_per_w)
    res = _sc_gather_wide(sup, table2, b_per_w, 2 * D).reshape(B, 2 * D)
    odd = (lab & 1) == 1
    return jnp.where(odd[:, None], res[:, D:], res[:, :D])
